# Initial kernel scaffold; baseline (speedup 1.0000x reference)
import functools

import jax
import jax.numpy as jnp
from jax import lax
from jax.experimental import pallas as pl
from jax.experimental.pallas import tpu as pltpu
from jax.experimental.pallas import tpu_sc as plsc

N = 10000
E = 320000
D = 128
DE = 16
G = 64
USR = 12

NC = 2
NS = 16
NW = NC * NS

_C = 80
_EPW = E // NW
_NCH = _EPW // _C
_RPT = N // NS
_ZR = 125


@functools.lru_cache(maxsize=None)
def _make_edge_pass(d):
  vpr = d // 16
  mesh = plsc.VectorSubcoreMesh(core_axis_name="c", subcore_axis_name="s")

  @functools.partial(
      pl.kernel,
      out_type=jax.ShapeDtypeStruct((NC, N, d), jnp.float32),
      mesh=mesh,
      scratch_types=[
          pltpu.VMEM((_C,), jnp.int32),
          pltpu.VMEM((_C,), jnp.int32),
          pltpu.VMEM((_C, d), jnp.float32),
          pltpu.VMEM((_C, d), jnp.float32),
          pltpu.VMEM((_ZR, d), jnp.float32),
          pltpu.VMEM_SHARED((N, d), jnp.float32),
          pltpu.SemaphoreType.DMA,
      ],
  )
  def edge_pass(x_hbm, e_hbm, src_hbm, dst_hbm, out_hbm,
                srcv, dstv, xrows, erows, zbuf, acc, sem):
    c = lax.axis_index("c")
    s = lax.axis_index("s")
    wid = c * NS + s

    zval = jnp.zeros((16,), jnp.float32)

    def zrow(r, carry):
      for j in range(vpr):
        zbuf[r, pl.ds(j * 16, 16)] = zval
      return carry

    lax.fori_loop(0, _ZR, zrow, 0)
    for k in range(_RPT // _ZR):
      pltpu.sync_copy(zbuf, acc.at[pl.ds(s * _RPT + k * _ZR, _ZR)])
    plsc.subcore_barrier()

    def chunk(k, carry):
      base = pl.multiple_of(wid * _EPW + k * _C, 8)
      pltpu.sync_copy(src_hbm.at[pl.ds(base, _C)], srcv)
      pltpu.sync_copy(dst_hbm.at[pl.ds(base, _C)], dstv)
      pltpu.sync_copy(e_hbm.at[pl.ds(base, _C)], erows)
      pltpu.async_copy(x_hbm.at[srcv], xrows, sem).wait()

      def row(r, rc):
        for j in range(vpr):
          sl = pl.ds(j * 16, 16)
          erows[r, sl] = jnp.maximum(erows[r, sl] + xrows[r, sl], 0.0)
        return rc

      lax.fori_loop(0, _C, row, 0)
      pltpu.sync_copy(erows, acc.at[dstv], add=True)
      return carry

    lax.fori_loop(0, _NCH, chunk, 0)
    plsc.subcore_barrier()

    for k in range(_RPT // _ZR):
      r0 = s * _RPT + k * _ZR
      pltpu.sync_copy(acc.at[pl.ds(r0, _ZR)], out_hbm.at[c, pl.ds(r0, _ZR)])

  return edge_pass


def _edge_embed_body(ea_ref, w1_ref, b1_ref, w2_ref, b2_ref, e1_ref, e2_ref):
  ea = ea_ref[...]
  e1_ref[...] = jnp.dot(ea, w1_ref[...],
                        preferred_element_type=jnp.float32) + b1_ref[...]
  e2_ref[...] = jnp.dot(ea, w2_ref[...],
                        preferred_element_type=jnp.float32) + b2_ref[...]


def _edge_embed(ea, w1, b1, w2, b2):
  be = 2000
  return pl.pallas_call(
      _edge_embed_body,
      grid=(E // be,),
      in_specs=[
          pl.BlockSpec((be, DE), lambda i: (i, 0)),
          pl.BlockSpec((DE, D), lambda i: (0, 0)),
          pl.BlockSpec((1, D), lambda i: (0, 0)),
          pl.BlockSpec((DE, 64), lambda i: (0, 0)),
          pl.BlockSpec((1, 64), lambda i: (0, 0)),
      ],
      out_specs=[
          pl.BlockSpec((be, D), lambda i: (i, 0)),
          pl.BlockSpec((be, 64), lambda i: (i, 0)),
      ],
      out_shape=[
          jax.ShapeDtypeStruct((E, D), jnp.float32),
          jax.ShapeDtypeStruct((E, 64), jnp.float32),
      ],
  )(ea, w1, b1.reshape(1, -1), w2, b2.reshape(1, -1))


def _make_node_mlp_body(relu_out):
  def body(x_ref, agg_ref, w1_ref, b1_ref, w2_ref, b2_ref, o_ref):
    h = x_ref[...] + agg_ref[0] + agg_ref[1]
    z = jnp.maximum(
        jnp.dot(h, w1_ref[...], preferred_element_type=jnp.float32)
        + b1_ref[...], 0.0)
    o = jnp.dot(z, w2_ref[...], preferred_element_type=jnp.float32) + b2_ref[...]
    if relu_out:
      o = jnp.maximum(o, 0.0)
    o_ref[...] = o
  return body


def _node_mlp(x, aggs, w1, b1, w2, b2, relu_out):
  bn = 2000
  din = x.shape[1]
  dh = w1.shape[1]
  dout = w2.shape[1]
  return pl.pallas_call(
      _make_node_mlp_body(relu_out),
      grid=(N // bn,),
      in_specs=[
          pl.BlockSpec((bn, din), lambda i: (i, 0)),
          pl.BlockSpec((NC, bn, din), lambda i: (0, i, 0)),
          pl.BlockSpec((din, dh), lambda i: (0, 0)),
          pl.BlockSpec((1, dh), lambda i: (0, 0)),
          pl.BlockSpec((dh, dout), lambda i: (0, 0)),
          pl.BlockSpec((1, dout), lambda i: (0, 0)),
      ],
      out_specs=pl.BlockSpec((bn, dout), lambda i: (i, 0)),
      out_shape=jax.ShapeDtypeStruct((N, dout), jnp.float32),
  )(x, aggs, w1, b1.reshape(1, -1), w2, b2.reshape(1, -1))


def _pool_body(nodes_ref, batch_ref, sums_ref, cnt_ref):
  i = pl.program_id(0)

  @pl.when(i == 0)
  def _():
    sums_ref[...] = jnp.zeros_like(sums_ref)
    cnt_ref[...] = jnp.zeros_like(cnt_ref)

  b = batch_ref[0, 0, :]
  bn = b.shape[0]
  oh = (b[None, :] == lax.broadcasted_iota(jnp.int32, (G, bn), 0)
        ).astype(jnp.float32)
  sums_ref[...] += jnp.dot(oh, nodes_ref[...],
                           preferred_element_type=jnp.float32)
  cnt_ref[...] += jnp.broadcast_to(jnp.sum(oh, axis=1)[:, None], (G, D))


def _pool(nodes, batch):
  bn = 1000
  nb = N // bn
  batch3 = batch.astype(jnp.int32).reshape(nb, 1, bn)
  return pl.pallas_call(
      _pool_body,
      grid=(nb,),
      in_specs=[
          pl.BlockSpec((bn, D), lambda i: (i, 0)),
          pl.BlockSpec((1, 1, bn), lambda i: (i, 0, 0)),
      ],
      out_specs=[
          pl.BlockSpec((G, D), lambda i: (0, 0)),
          pl.BlockSpec((G, D), lambda i: (0, 0)),
      ],
      out_shape=[
          jax.ShapeDtypeStruct((G, D), jnp.float32),
          jax.ShapeDtypeStruct((G, D), jnp.float32),
      ],
  )(nodes, batch3)


def _head_body(sums_ref, cnt_ref, usr_ref, m1a_ref, m1u_ref, b1_ref,
               m2_ref, b2_ref, m3_ref, b3_ref, m4_ref, b4_ref,
               m5_ref, b5_ref, o_ref):
  pooled = sums_ref[...] / jnp.maximum(cnt_ref[...], 1.0)
  z = jnp.maximum(
      jnp.dot(pooled, m1a_ref[...], preferred_element_type=jnp.float32)
      + jnp.dot(usr_ref[...], m1u_ref[...], preferred_element_type=jnp.float32)
      + b1_ref[...], 0.0)
  z = jnp.maximum(
      jnp.dot(z, m2_ref[...], preferred_element_type=jnp.float32)
      + b2_ref[...], 0.0)
  z = jnp.maximum(
      jnp.dot(z, m3_ref[...], preferred_element_type=jnp.float32)
      + b3_ref[...], 0.0)
  z = jnp.maximum(
      jnp.dot(z, m4_ref[...], preferred_element_type=jnp.float32)
      + b4_ref[...], 0.0)
  o_ref[...] = jnp.dot(z, m5_ref[...],
                       preferred_element_type=jnp.float32) + b5_ref[...]


def _head(sums, cnt, usr, m1_W, m1_b, m2_W, m2_b, m3_W, m3_b,
          m4_W, m4_b, m5_W, m5_b):
  m1a = m1_W[:D]
  m1u = m1_W[D:]
  args = (sums, cnt, usr, m1a, m1u, m1_b.reshape(1, -1),
          m2_W, m2_b.reshape(1, -1), m3_W, m3_b.reshape(1, -1),
          m4_W, m4_b.reshape(1, -1), m5_W, m5_b.reshape(1, -1))
  return pl.pallas_call(
      _head_body,
      out_shape=jax.ShapeDtypeStruct((G, 1), jnp.float32),
  )(*args)


def kernel(x, edge_index, edge_attr, batch, usr,
           lin_e1_W, lin_e1_b, nn1_W1, nn1_b1, nn1_W2, nn1_b2,
           lin_e2_W, lin_e2_b, nn2_W1, nn2_b1, nn2_W2, nn2_b2,
           m1_W, m1_b, m2_W, m2_b, m3_W, m3_b, m4_W, m4_b, m5_W, m5_b):
  src = edge_index[0].astype(jnp.int32)
  dst = edge_index[1].astype(jnp.int32)

  e1, e2 = _edge_embed(edge_attr, lin_e1_W, lin_e1_b, lin_e2_W, lin_e2_b)

  agg1 = _make_edge_pass(D)(x, e1, src, dst)
  h1 = _node_mlp(x, agg1, nn1_W1, nn1_b1, nn1_W2, nn1_b2, relu_out=True)

  agg2 = _make_edge_pass(64)(h1, e2, src, dst)
  h2 = _node_mlp(h1, agg2, nn2_W1, nn2_b1, nn2_W2, nn2_b2, relu_out=False)

  sums, cnt = _pool(h2, batch)
  return _head(sums, cnt, usr, m1_W, m1_b, m2_W, m2_b, m3_W, m3_b,
               m4_W, m4_b, m5_W, m5_b)


# speedup vs baseline: 2.5541x; 2.5541x over previous
import functools

import jax
import jax.numpy as jnp
from jax import lax
from jax.experimental import pallas as pl
from jax.experimental.pallas import tpu as pltpu
from jax.experimental.pallas import tpu_sc as plsc

N = 10000
E = 320000
D = 128
DE = 16
G = 64
USR = 12

NC = 2
NS = 16
NW = NC * NS

_C = 80
_EPW = E // NW
_NCH = _EPW // _C
_RST = 624
_RSZ = 640
_ZR = 160


@functools.lru_cache(maxsize=None)
def _make_edge_pass(d):
  vpr = d // 16
  mesh = plsc.VectorSubcoreMesh(core_axis_name="c", subcore_axis_name="s")

  @functools.partial(
      pl.kernel,
      out_type=jax.ShapeDtypeStruct((NC, N, d), jnp.float32),
      mesh=mesh,
      scratch_types=[
          pltpu.VMEM((_C,), jnp.int32),
          pltpu.VMEM((_C,), jnp.int32),
          pltpu.VMEM((_C, d), jnp.float32),
          pltpu.VMEM((_C, d), jnp.float32),
          pltpu.VMEM((_ZR, d), jnp.float32),
          pltpu.VMEM_SHARED((N, d), jnp.float32),
          pltpu.SemaphoreType.DMA,
      ],
      compiler_params=pltpu.CompilerParams(use_tc_tiling_on_sc=False),
  )
  def edge_pass(x_hbm, e_hbm, src_hbm, dst_hbm, out_hbm,
                srcv, dstv, xrows, erows, zbuf, acc, sem):
    c = lax.axis_index("c")
    s = lax.axis_index("s")
    wid = c * NS + s

    zval = jnp.zeros((16,), jnp.float32)

    def zrow(r, carry):
      for j in range(vpr):
        zbuf[r, pl.ds(j * 16, 16)] = zval
      return carry

    lax.fori_loop(0, _ZR, zrow, 0)
    for k in range(_RSZ // _ZR):
      pltpu.sync_copy(zbuf, acc.at[pl.ds(s * _RST + k * _ZR, _ZR)])
    plsc.subcore_barrier()

    def chunk(k, carry):
      base = pl.multiple_of(wid * _EPW + k * _C, 8)
      pltpu.sync_copy(src_hbm.at[pl.ds(base, _C)], srcv)
      pltpu.sync_copy(dst_hbm.at[pl.ds(base, _C)], dstv)
      pltpu.sync_copy(e_hbm.at[pl.ds(base, _C)], erows)
      pltpu.async_copy(x_hbm.at[srcv], xrows, sem).wait()

      def row(r, rc):
        for j in range(vpr):
          sl = pl.ds(j * 16, 16)
          erows[r, sl] = jnp.maximum(erows[r, sl] + xrows[r, sl], 0.0)
        return rc

      lax.fori_loop(0, _C, row, 0)
      pltpu.sync_copy(erows, acc.at[dstv], add=True)
      return carry

    lax.fori_loop(0, _NCH, chunk, 0)
    plsc.subcore_barrier()

    for k in range(_RSZ // _ZR):
      r0 = s * _RST + k * _ZR
      pltpu.sync_copy(acc.at[pl.ds(r0, _ZR)], out_hbm.at[c, pl.ds(r0, _ZR)])

  return edge_pass


def _edge_embed_body(ea_ref, w1_ref, b1_ref, w2_ref, b2_ref, e1_ref, e2_ref):
  ea = ea_ref[...]
  e1_ref[...] = jnp.dot(ea, w1_ref[...],
                        preferred_element_type=jnp.float32) + b1_ref[...]
  e2_ref[...] = jnp.dot(ea, w2_ref[...],
                        preferred_element_type=jnp.float32) + b2_ref[...]


def _edge_embed(ea, w1, b1, w2, b2):
  be = 2000
  return pl.pallas_call(
      _edge_embed_body,
      grid=(E // be,),
      in_specs=[
          pl.BlockSpec((be, DE), lambda i: (i, 0)),
          pl.BlockSpec((DE, D), lambda i: (0, 0)),
          pl.BlockSpec((1, D), lambda i: (0, 0)),
          pl.BlockSpec((DE, 64), lambda i: (0, 0)),
          pl.BlockSpec((1, 64), lambda i: (0, 0)),
      ],
      out_specs=[
          pl.BlockSpec((be, D), lambda i: (i, 0)),
          pl.BlockSpec((be, 64), lambda i: (i, 0)),
      ],
      out_shape=[
          jax.ShapeDtypeStruct((E, D), jnp.float32),
          jax.ShapeDtypeStruct((E, 64), jnp.float32),
      ],
  )(ea, w1, b1.reshape(1, -1), w2, b2.reshape(1, -1))


def _make_node_mlp_body(relu_out):
  def body(x_ref, agg_ref, w1_ref, b1_ref, w2_ref, b2_ref, o_ref):
    h = x_ref[...] + agg_ref[0] + agg_ref[1]
    z = jnp.maximum(
        jnp.dot(h, w1_ref[...], preferred_element_type=jnp.float32)
        + b1_ref[...], 0.0)
    o = jnp.dot(z, w2_ref[...], preferred_element_type=jnp.float32) + b2_ref[...]
    if relu_out:
      o = jnp.maximum(o, 0.0)
    o_ref[...] = o
  return body


def _node_mlp(x, aggs, w1, b1, w2, b2, relu_out):
  bn = 2000
  din = x.shape[1]
  dh = w1.shape[1]
  dout = w2.shape[1]
  return pl.pallas_call(
      _make_node_mlp_body(relu_out),
      grid=(N // bn,),
      in_specs=[
          pl.BlockSpec((bn, din), lambda i: (i, 0)),
          pl.BlockSpec((NC, bn, din), lambda i: (0, i, 0)),
          pl.BlockSpec((din, dh), lambda i: (0, 0)),
          pl.BlockSpec((1, dh), lambda i: (0, 0)),
          pl.BlockSpec((dh, dout), lambda i: (0, 0)),
          pl.BlockSpec((1, dout), lambda i: (0, 0)),
      ],
      out_specs=pl.BlockSpec((bn, dout), lambda i: (i, 0)),
      out_shape=jax.ShapeDtypeStruct((N, dout), jnp.float32),
  )(x, aggs, w1, b1.reshape(1, -1), w2, b2.reshape(1, -1))


def _pool_body(nodes_ref, batch_ref, sums_ref, cnt_ref):
  i = pl.program_id(0)

  @pl.when(i == 0)
  def _():
    sums_ref[...] = jnp.zeros_like(sums_ref)
    cnt_ref[...] = jnp.zeros_like(cnt_ref)

  b = batch_ref[0, 0, :]
  bn = b.shape[0]
  oh = (b[None, :] == lax.broadcasted_iota(jnp.int32, (G, bn), 0)
        ).astype(jnp.float32)
  sums_ref[...] += jnp.dot(oh, nodes_ref[...],
                           preferred_element_type=jnp.float32,
                           precision=lax.Precision.HIGHEST)
  cnt_ref[...] += jnp.broadcast_to(jnp.sum(oh, axis=1)[:, None], (G, D))


def _pool(nodes, batch):
  bn = 1000
  nb = N // bn
  batch3 = batch.astype(jnp.int32).reshape(nb, 1, bn)
  return pl.pallas_call(
      _pool_body,
      grid=(nb,),
      in_specs=[
          pl.BlockSpec((bn, D), lambda i: (i, 0)),
          pl.BlockSpec((1, 1, bn), lambda i: (i, 0, 0)),
      ],
      out_specs=[
          pl.BlockSpec((G, D), lambda i: (0, 0)),
          pl.BlockSpec((G, D), lambda i: (0, 0)),
      ],
      out_shape=[
          jax.ShapeDtypeStruct((G, D), jnp.float32),
          jax.ShapeDtypeStruct((G, D), jnp.float32),
      ],
  )(nodes, batch3)


def _head_body(sums_ref, cnt_ref, usr_ref, m1a_ref, m1u_ref, b1_ref,
               m2_ref, b2_ref, m3_ref, b3_ref, m4_ref, b4_ref,
               m5_ref, b5_ref, o_ref):
  pooled = sums_ref[...] / jnp.maximum(cnt_ref[...], 1.0)
  z = jnp.maximum(
      jnp.dot(pooled, m1a_ref[...], preferred_element_type=jnp.float32)
      + jnp.dot(usr_ref[...], m1u_ref[...], preferred_element_type=jnp.float32)
      + b1_ref[...], 0.0)
  z = jnp.maximum(
      jnp.dot(z, m2_ref[...], preferred_element_type=jnp.float32)
      + b2_ref[...], 0.0)
  z = jnp.maximum(
      jnp.dot(z, m3_ref[...], preferred_element_type=jnp.float32)
      + b3_ref[...], 0.0)
  z = jnp.maximum(
      jnp.dot(z, m4_ref[...], preferred_element_type=jnp.float32)
      + b4_ref[...], 0.0)
  o_ref[...] = jnp.dot(z, m5_ref[...],
                       preferred_element_type=jnp.float32) + b5_ref[...]


def _head(sums, cnt, usr, m1_W, m1_b, m2_W, m2_b, m3_W, m3_b,
          m4_W, m4_b, m5_W, m5_b):
  m1a = m1_W[:D]
  m1u = m1_W[D:]
  args = (sums, cnt, usr, m1a, m1u, m1_b.reshape(1, -1),
          m2_W, m2_b.reshape(1, -1), m3_W, m3_b.reshape(1, -1),
          m4_W, m4_b.reshape(1, -1), m5_W, m5_b.reshape(1, -1))
  return pl.pallas_call(
      _head_body,
      out_shape=jax.ShapeDtypeStruct((G, 1), jnp.float32),
  )(*args)


def kernel(x, edge_index, edge_attr, batch, usr,
           lin_e1_W, lin_e1_b, nn1_W1, nn1_b1, nn1_W2, nn1_b2,
           lin_e2_W, lin_e2_b, nn2_W1, nn2_b1, nn2_W2, nn2_b2,
           m1_W, m1_b, m2_W, m2_b, m3_W, m3_b, m4_W, m4_b, m5_W, m5_b):
  src = edge_index[0].astype(jnp.int32)
  dst = edge_index[1].astype(jnp.int32)

  e1, e2 = _edge_embed(edge_attr, lin_e1_W, lin_e1_b, lin_e2_W, lin_e2_b)

  agg1 = _make_edge_pass(D)(x, e1, src, dst)
  h1 = _node_mlp(x, agg1, nn1_W1, nn1_b1, nn1_W2, nn1_b2, relu_out=True)

  agg2 = _make_edge_pass(64)(h1, e2, src, dst)
  h2 = _node_mlp(h1, agg2, nn2_W1, nn2_b1, nn2_W2, nn2_b2, relu_out=False)

  sums, cnt = _pool(h2, batch)
  return _head(sums, cnt, usr, m1_W, m1_b, m2_W, m2_b, m3_W, m3_b,
               m4_W, m4_b, m5_W, m5_b)


# speedup vs baseline: 3.8586x; 1.5108x over previous
import functools

import jax
import jax.numpy as jnp
from jax import lax
from jax.experimental import pallas as pl
from jax.experimental.pallas import tpu as pltpu
from jax.experimental.pallas import tpu_sc as plsc

N = 10000
E = 320000
D = 128
DE = 16
G = 64
USR = 12

NC = 2
NS = 16
NW = NC * NS

_C = 80
_EPW = E // NW
_NCH = _EPW // _C
_RST = 624
_RSZ = 640
_ZR = 160


@functools.lru_cache(maxsize=None)
def _make_edge_pass(d):
  vpr = d // 16
  mesh = plsc.VectorSubcoreMesh(core_axis_name="c", subcore_axis_name="s")

  @functools.partial(
      pl.kernel,
      out_type=jax.ShapeDtypeStruct((NC, N, d), jnp.float32),
      mesh=mesh,
      scratch_types=[
          pltpu.VMEM((2, _C), jnp.int32),
          pltpu.VMEM((2, _C), jnp.int32),
          pltpu.VMEM((2, _C, d), jnp.float32),
          pltpu.VMEM((2, _C, d), jnp.float32),
          pltpu.VMEM_SHARED((N, d), jnp.float32),
          pltpu.SemaphoreType.DMA,
          pltpu.SemaphoreType.DMA,
          pltpu.SemaphoreType.DMA,
          pltpu.SemaphoreType.DMA,
      ],
      compiler_params=pltpu.CompilerParams(use_tc_tiling_on_sc=False),
  )
  def edge_pass(x_hbm, e_hbm, src_hbm, dst_hbm, out_hbm,
                srcv, dstv, xrows, erows, acc,
                asem0, asem1, gsem0, gsem1):
    c = lax.axis_index("c")
    s = lax.axis_index("s")
    wid = c * NS + s
    asem = (asem0, asem1)
    gsem = (gsem0, gsem1)

    zval = jnp.zeros((16,), jnp.float32)

    def zrow(r, carry):
      for j in range(vpr):
        erows[0, r, pl.ds(j * 16, 16)] = zval
      return carry

    lax.fori_loop(0, _C, zrow, 0)
    for k in range(_RSZ // _C):
      pltpu.sync_copy(erows.at[0], acc.at[pl.ds(s * _RST + k * _C, _C)])
    plsc.subcore_barrier()

    def ebase(k):
      return pl.multiple_of(wid * _EPW + k * _C, 8)

    def issue_linear(k, b):
      base = ebase(k)
      pltpu.async_copy(src_hbm.at[pl.ds(base, _C)], srcv.at[b], asem[b])
      pltpu.async_copy(dst_hbm.at[pl.ds(base, _C)], dstv.at[b], asem[b])
      pltpu.async_copy(e_hbm.at[pl.ds(base, _C)], erows.at[b], asem[b])

    def drain_linear(k, b):
      base = ebase(k)
      pltpu.make_async_copy(src_hbm.at[pl.ds(base, _C)], srcv.at[b], asem[b]).wait()
      pltpu.make_async_copy(dst_hbm.at[pl.ds(base, _C)], dstv.at[b], asem[b]).wait()
      pltpu.make_async_copy(e_hbm.at[pl.ds(base, _C)], erows.at[b], asem[b]).wait()

    def issue_gather(b):
      pltpu.async_copy(x_hbm.at[srcv.at[b]], xrows.at[b], gsem[b])

    def drain_gather(b):
      pltpu.make_async_copy(x_hbm.at[srcv.at[b]], xrows.at[b], gsem[b]).wait()

    def step(k, b, last):
      drain_gather(b)
      if not last:
        drain_linear(k + 1, 1 - b)
        issue_gather(1 - b)

      def row(r, rc):
        for j in range(vpr):
          sl = pl.ds(j * 16, 16)
          erows[b, r, sl] = jnp.maximum(erows[b, r, sl] + xrows[b, r, sl], 0.0)
        return rc

      lax.fori_loop(0, _C, row, 0)
      pltpu.sync_copy(erows.at[b], acc.at[dstv.at[b]], add=True)
      if not last:
        @pl.when(k + 2 < _NCH)
        def _():
          issue_linear(k + 2, b)

    issue_linear(0, 0)
    drain_linear(0, 0)
    issue_gather(0)
    issue_linear(1, 1)

    def pair(t, carry):
      step(2 * t, 0, last=False)
      step(2 * t + 1, 1, last=False)
      return carry

    lax.fori_loop(0, (_NCH - 1) // 2, pair, 0)
    step(_NCH - 1, (_NCH - 1) % 2, last=True)

    plsc.subcore_barrier()
    for k in range(_RSZ // _ZR):
      r0 = s * _RST + k * _ZR
      pltpu.sync_copy(acc.at[pl.ds(r0, _ZR)], out_hbm.at[c, pl.ds(r0, _ZR)])


  return edge_pass


def _edge_embed_body(ea_ref, w1_ref, b1_ref, w2_ref, b2_ref, e1_ref, e2_ref):
  ea = ea_ref[...]
  e1_ref[...] = jnp.dot(ea, w1_ref[...],
                        preferred_element_type=jnp.float32) + b1_ref[...]
  e2_ref[...] = jnp.dot(ea, w2_ref[...],
                        preferred_element_type=jnp.float32) + b2_ref[...]


def _edge_embed(ea, w1, b1, w2, b2):
  be = 2000
  return pl.pallas_call(
      _edge_embed_body,
      grid=(E // be,),
      in_specs=[
          pl.BlockSpec((be, DE), lambda i: (i, 0)),
          pl.BlockSpec((DE, D), lambda i: (0, 0)),
          pl.BlockSpec((1, D), lambda i: (0, 0)),
          pl.BlockSpec((DE, 64), lambda i: (0, 0)),
          pl.BlockSpec((1, 64), lambda i: (0, 0)),
      ],
      out_specs=[
          pl.BlockSpec((be, D), lambda i: (i, 0)),
          pl.BlockSpec((be, 64), lambda i: (i, 0)),
      ],
      out_shape=[
          jax.ShapeDtypeStruct((E, D), jnp.float32),
          jax.ShapeDtypeStruct((E, 64), jnp.float32),
      ],
  )(ea, w1, b1.reshape(1, -1), w2, b2.reshape(1, -1))


def _make_node_mlp_body(relu_out):
  def body(x_ref, agg_ref, w1_ref, b1_ref, w2_ref, b2_ref, o_ref):
    h = x_ref[...] + agg_ref[0] + agg_ref[1]
    z = jnp.maximum(
        jnp.dot(h, w1_ref[...], preferred_element_type=jnp.float32)
        + b1_ref[...], 0.0)
    o = jnp.dot(z, w2_ref[...], preferred_element_type=jnp.float32) + b2_ref[...]
    if relu_out:
      o = jnp.maximum(o, 0.0)
    o_ref[...] = o
  return body


def _node_mlp(x, aggs, w1, b1, w2, b2, relu_out):
  bn = 2000
  din = x.shape[1]
  dh = w1.shape[1]
  dout = w2.shape[1]
  return pl.pallas_call(
      _make_node_mlp_body(relu_out),
      grid=(N // bn,),
      in_specs=[
          pl.BlockSpec((bn, din), lambda i: (i, 0)),
          pl.BlockSpec((NC, bn, din), lambda i: (0, i, 0)),
          pl.BlockSpec((din, dh), lambda i: (0, 0)),
          pl.BlockSpec((1, dh), lambda i: (0, 0)),
          pl.BlockSpec((dh, dout), lambda i: (0, 0)),
          pl.BlockSpec((1, dout), lambda i: (0, 0)),
      ],
      out_specs=pl.BlockSpec((bn, dout), lambda i: (i, 0)),
      out_shape=jax.ShapeDtypeStruct((N, dout), jnp.float32),
  )(x, aggs, w1, b1.reshape(1, -1), w2, b2.reshape(1, -1))


def _pool_body(nodes_ref, batch_ref, sums_ref, cnt_ref):
  i = pl.program_id(0)

  @pl.when(i == 0)
  def _():
    sums_ref[...] = jnp.zeros_like(sums_ref)
    cnt_ref[...] = jnp.zeros_like(cnt_ref)

  b = batch_ref[0, 0, :]
  bn = b.shape[0]
  oh = (b[None, :] == lax.broadcasted_iota(jnp.int32, (G, bn), 0)
        ).astype(jnp.float32)
  sums_ref[...] += jnp.dot(oh, nodes_ref[...],
                           preferred_element_type=jnp.float32,
                           precision=lax.Precision.HIGHEST)
  cnt_ref[...] += jnp.broadcast_to(jnp.sum(oh, axis=1)[:, None], (G, D))


def _pool(nodes, batch):
  bn = 1000
  nb = N // bn
  batch3 = batch.astype(jnp.int32).reshape(nb, 1, bn)
  return pl.pallas_call(
      _pool_body,
      grid=(nb,),
      in_specs=[
          pl.BlockSpec((bn, D), lambda i: (i, 0)),
          pl.BlockSpec((1, 1, bn), lambda i: (i, 0, 0)),
      ],
      out_specs=[
          pl.BlockSpec((G, D), lambda i: (0, 0)),
          pl.BlockSpec((G, D), lambda i: (0, 0)),
      ],
      out_shape=[
          jax.ShapeDtypeStruct((G, D), jnp.float32),
          jax.ShapeDtypeStruct((G, D), jnp.float32),
      ],
  )(nodes, batch3)


def _head_body(sums_ref, cnt_ref, usr_ref, m1a_ref, m1u_ref, b1_ref,
               m2_ref, b2_ref, m3_ref, b3_ref, m4_ref, b4_ref,
               m5_ref, b5_ref, o_ref):
  pooled = sums_ref[...] / jnp.maximum(cnt_ref[...], 1.0)
  z = jnp.maximum(
      jnp.dot(pooled, m1a_ref[...], preferred_element_type=jnp.float32)
      + jnp.dot(usr_ref[...], m1u_ref[...], preferred_element_type=jnp.float32)
      + b1_ref[...], 0.0)
  z = jnp.maximum(
      jnp.dot(z, m2_ref[...], preferred_element_type=jnp.float32)
      + b2_ref[...], 0.0)
  z = jnp.maximum(
      jnp.dot(z, m3_ref[...], preferred_element_type=jnp.float32)
      + b3_ref[...], 0.0)
  z = jnp.maximum(
      jnp.dot(z, m4_ref[...], preferred_element_type=jnp.float32)
      + b4_ref[...], 0.0)
  o_ref[...] = jnp.dot(z, m5_ref[...],
                       preferred_element_type=jnp.float32) + b5_ref[...]


def _head(sums, cnt, usr, m1_W, m1_b, m2_W, m2_b, m3_W, m3_b,
          m4_W, m4_b, m5_W, m5_b):
  m1a = m1_W[:D]
  m1u = m1_W[D:]
  args = (sums, cnt, usr, m1a, m1u, m1_b.reshape(1, -1),
          m2_W, m2_b.reshape(1, -1), m3_W, m3_b.reshape(1, -1),
          m4_W, m4_b.reshape(1, -1), m5_W, m5_b.reshape(1, -1))
  return pl.pallas_call(
      _head_body,
      out_shape=jax.ShapeDtypeStruct((G, 1), jnp.float32),
  )(*args)


def kernel(x, edge_index, edge_attr, batch, usr,
           lin_e1_W, lin_e1_b, nn1_W1, nn1_b1, nn1_W2, nn1_b2,
           lin_e2_W, lin_e2_b, nn2_W1, nn2_b1, nn2_W2, nn2_b2,
           m1_W, m1_b, m2_W, m2_b, m3_W, m3_b, m4_W, m4_b, m5_W, m5_b):
  src = edge_index[0].astype(jnp.int32)
  dst = edge_index[1].astype(jnp.int32)

  e1, e2 = _edge_embed(edge_attr, lin_e1_W, lin_e1_b, lin_e2_W, lin_e2_b)

  agg1 = _make_edge_pass(D)(x, e1, src, dst)
  h1 = _node_mlp(x, agg1, nn1_W1, nn1_b1, nn1_W2, nn1_b2, relu_out=True)

  agg2 = _make_edge_pass(64)(h1, e2, src, dst)
  h2 = _node_mlp(h1, agg2, nn2_W1, nn2_b1, nn2_W2, nn2_b2, relu_out=False)

  sums, cnt = _pool(h2, batch)
  return _head(sums, cnt, usr, m1_W, m1_b, m2_W, m2_b, m3_W, m3_b,
               m4_W, m4_b, m5_W, m5_b)


# speedup vs baseline: 3.9074x; 1.0126x over previous
import functools

import jax
import jax.numpy as jnp
from jax import lax
from jax.experimental import pallas as pl
from jax.experimental.pallas import tpu as pltpu
from jax.experimental.pallas import tpu_sc as plsc

N = 10000
E = 320000
D = 128
DE = 16
G = 64
USR = 12

NC = 2
NS = 16
NW = NC * NS

_C = 80
_EPW = E // NW
_NCH = _EPW // _C
_RST = 624
_RSZ = 640
_ZR = 160


@functools.lru_cache(maxsize=None)
def _make_edge_pass(d):
  vpr = d // 16
  mesh = plsc.VectorSubcoreMesh(core_axis_name="c", subcore_axis_name="s")

  @functools.partial(
      pl.kernel,
      out_type=jax.ShapeDtypeStruct((NC, N, d), jnp.float32),
      mesh=mesh,
      scratch_types=[
          pltpu.VMEM((2, _C), jnp.int32),
          pltpu.VMEM((2, _C), jnp.int32),
          pltpu.VMEM((2, _C, d), jnp.float32),
          pltpu.VMEM((2, _C, d), jnp.float32),
          pltpu.VMEM_SHARED((N, d), jnp.float32),
          pltpu.SemaphoreType.DMA,
          pltpu.SemaphoreType.DMA,
          pltpu.SemaphoreType.DMA,
          pltpu.SemaphoreType.DMA,
      ],
      compiler_params=pltpu.CompilerParams(use_tc_tiling_on_sc=False),
  )
  def edge_pass(x_hbm, e_hbm, src_hbm, dst_hbm, out_hbm,
                srcv, dstv, xrows, erows, acc,
                asem0, asem1, gsem0, gsem1):
    c = lax.axis_index("c")
    s = lax.axis_index("s")
    wid = c * NS + s
    asem = (asem0, asem1)
    gsem = (gsem0, gsem1)

    zval = jnp.zeros((16,), jnp.float32)

    def zrow(r, carry):
      for j in range(vpr):
        erows[0, r, pl.ds(j * 16, 16)] = zval
      return carry

    lax.fori_loop(0, _C, zrow, 0)
    for k in range(_RSZ // _C):
      pltpu.sync_copy(erows.at[0], acc.at[pl.ds(s * _RST + k * _C, _C)])
    plsc.subcore_barrier()

    def ebase(k):
      return pl.multiple_of(wid * _EPW + k * _C, 8)

    def issue_linear(k, b):
      base = ebase(k)
      pltpu.async_copy(src_hbm.at[pl.ds(base, _C)], srcv.at[b], asem[b])
      pltpu.async_copy(dst_hbm.at[pl.ds(base, _C)], dstv.at[b], asem[b])
      pltpu.async_copy(e_hbm.at[pl.ds(base, _C)], erows.at[b], asem[b])

    def drain_linear(k, b):
      base = ebase(k)
      pltpu.make_async_copy(src_hbm.at[pl.ds(base, _C)], srcv.at[b], asem[b]).wait()
      pltpu.make_async_copy(dst_hbm.at[pl.ds(base, _C)], dstv.at[b], asem[b]).wait()
      pltpu.make_async_copy(e_hbm.at[pl.ds(base, _C)], erows.at[b], asem[b]).wait()

    def issue_gather(b):
      pltpu.async_copy(x_hbm.at[srcv.at[b]], xrows.at[b], gsem[b])

    def drain_gather(b):
      pltpu.make_async_copy(x_hbm.at[srcv.at[b]], xrows.at[b], gsem[b]).wait()

    def step(k, b, last):
      drain_gather(b)
      if not last:
        drain_linear(k + 1, 1 - b)
        issue_gather(1 - b)

      def row(r, rc):
        for j in range(vpr):
          sl = pl.ds(j * 16, 16)
          erows[b, r, sl] = jnp.maximum(erows[b, r, sl] + xrows[b, r, sl], 0.0)
        return rc

      lax.fori_loop(0, _C, row, 0)
      pltpu.sync_copy(erows.at[b], acc.at[dstv.at[b]], add=True)
      if not last:
        @pl.when(k + 2 < _NCH)
        def _():
          issue_linear(k + 2, b)

    issue_linear(0, 0)
    drain_linear(0, 0)
    issue_gather(0)
    issue_linear(1, 1)

    def pair(t, carry):
      step(2 * t, 0, last=False)
      step(2 * t + 1, 1, last=False)
      return carry

    lax.fori_loop(0, (_NCH - 1) // 2, pair, 0)
    step(_NCH - 1, (_NCH - 1) % 2, last=True)

    plsc.subcore_barrier()
    for k in range(_RSZ // _ZR):
      r0 = s * _RST + k * _ZR
      pltpu.sync_copy(acc.at[pl.ds(r0, _ZR)], out_hbm.at[c, pl.ds(r0, _ZR)])


  return edge_pass


def _edge_embed_body(ea_ref, w_ref, b_ref, e_ref):
  e_ref[...] = jnp.dot(ea_ref[...], w_ref[...],
                       preferred_element_type=jnp.float32) + b_ref[...]


def _edge_embed(ea, w, b):
  be = 2000
  dout = w.shape[1]
  return pl.pallas_call(
      _edge_embed_body,
      grid=(E // be,),
      in_specs=[
          pl.BlockSpec((be, DE), lambda i: (i, 0)),
          pl.BlockSpec((DE, dout), lambda i: (0, 0)),
          pl.BlockSpec((1, dout), lambda i: (0, 0)),
      ],
      out_specs=pl.BlockSpec((be, dout), lambda i: (i, 0)),
      out_shape=jax.ShapeDtypeStruct((E, dout), jnp.float32),
  )(ea, w, b.reshape(1, -1))


def _make_node_mlp_body(relu_out):
  def body(x_ref, agg_ref, w1_ref, b1_ref, w2_ref, b2_ref, o_ref):
    h = x_ref[...] + agg_ref[0] + agg_ref[1]
    z = jnp.maximum(
        jnp.dot(h, w1_ref[...], preferred_element_type=jnp.float32)
        + b1_ref[...], 0.0)
    o = jnp.dot(z, w2_ref[...], preferred_element_type=jnp.float32) + b2_ref[...]
    if relu_out:
      o = jnp.maximum(o, 0.0)
    o_ref[...] = o
  return body


def _node_mlp(x, aggs, w1, b1, w2, b2, relu_out):
  bn = 2000
  din = x.shape[1]
  dh = w1.shape[1]
  dout = w2.shape[1]
  return pl.pallas_call(
      _make_node_mlp_body(relu_out),
      grid=(N // bn,),
      in_specs=[
          pl.BlockSpec((bn, din), lambda i: (i, 0)),
          pl.BlockSpec((NC, bn, din), lambda i: (0, i, 0)),
          pl.BlockSpec((din, dh), lambda i: (0, 0)),
          pl.BlockSpec((1, dh), lambda i: (0, 0)),
          pl.BlockSpec((dh, dout), lambda i: (0, 0)),
          pl.BlockSpec((1, dout), lambda i: (0, 0)),
      ],
      out_specs=pl.BlockSpec((bn, dout), lambda i: (i, 0)),
      out_shape=jax.ShapeDtypeStruct((N, dout), jnp.float32),
  )(x, aggs, w1, b1.reshape(1, -1), w2, b2.reshape(1, -1))


def _pool_body(nodes_ref, batch_ref, sums_ref, cnt_ref):
  i = pl.program_id(0)

  @pl.when(i == 0)
  def _():
    sums_ref[...] = jnp.zeros_like(sums_ref)
    cnt_ref[...] = jnp.zeros_like(cnt_ref)

  b = batch_ref[0, 0, :]
  bn = b.shape[0]
  oh = (b[None, :] == lax.broadcasted_iota(jnp.int32, (G, bn), 0)
        ).astype(jnp.float32)
  sums_ref[...] += jnp.dot(oh, nodes_ref[...],
                           preferred_element_type=jnp.float32,
                           precision=lax.Precision.HIGHEST)
  cnt_ref[...] += jnp.broadcast_to(jnp.sum(oh, axis=1)[:, None], (G, D))


def _pool(nodes, batch):
  bn = 1000
  nb = N // bn
  batch3 = batch.astype(jnp.int32).reshape(nb, 1, bn)
  return pl.pallas_call(
      _pool_body,
      grid=(nb,),
      in_specs=[
          pl.BlockSpec((bn, D), lambda i: (i, 0)),
          pl.BlockSpec((1, 1, bn), lambda i: (i, 0, 0)),
      ],
      out_specs=[
          pl.BlockSpec((G, D), lambda i: (0, 0)),
          pl.BlockSpec((G, D), lambda i: (0, 0)),
      ],
      out_shape=[
          jax.ShapeDtypeStruct((G, D), jnp.float32),
          jax.ShapeDtypeStruct((G, D), jnp.float32),
      ],
  )(nodes, batch3)


def _head_body(sums_ref, cnt_ref, usr_ref, m1a_ref, m1u_ref, b1_ref,
               m2_ref, b2_ref, m3_ref, b3_ref, m4_ref, b4_ref,
               m5_ref, b5_ref, o_ref):
  pooled = sums_ref[...] / jnp.maximum(cnt_ref[...], 1.0)
  z = jnp.maximum(
      jnp.dot(pooled, m1a_ref[...], preferred_element_type=jnp.float32)
      + jnp.dot(usr_ref[...], m1u_ref[...], preferred_element_type=jnp.float32)
      + b1_ref[...], 0.0)
  z = jnp.maximum(
      jnp.dot(z, m2_ref[...], preferred_element_type=jnp.float32)
      + b2_ref[...], 0.0)
  z = jnp.maximum(
      jnp.dot(z, m3_ref[...], preferred_element_type=jnp.float32)
      + b3_ref[...], 0.0)
  z = jnp.maximum(
      jnp.dot(z, m4_ref[...], preferred_element_type=jnp.float32)
      + b4_ref[...], 0.0)
  o_ref[...] = jnp.dot(z, m5_ref[...],
                       preferred_element_type=jnp.float32) + b5_ref[...]


def _head(sums, cnt, usr, m1_W, m1_b, m2_W, m2_b, m3_W, m3_b,
          m4_W, m4_b, m5_W, m5_b):
  m1a = m1_W[:D]
  m1u = m1_W[D:]
  args = (sums, cnt, usr, m1a, m1u, m1_b.reshape(1, -1),
          m2_W, m2_b.reshape(1, -1), m3_W, m3_b.reshape(1, -1),
          m4_W, m4_b.reshape(1, -1), m5_W, m5_b.reshape(1, -1))
  return pl.pallas_call(
      _head_body,
      out_shape=jax.ShapeDtypeStruct((G, 1), jnp.float32),
  )(*args)


def kernel(x, edge_index, edge_attr, batch, usr,
           lin_e1_W, lin_e1_b, nn1_W1, nn1_b1, nn1_W2, nn1_b2,
           lin_e2_W, lin_e2_b, nn2_W1, nn2_b1, nn2_W2, nn2_b2,
           m1_W, m1_b, m2_W, m2_b, m3_W, m3_b, m4_W, m4_b, m5_W, m5_b):
  src = edge_index[0].astype(jnp.int32)
  dst = edge_index[1].astype(jnp.int32)

  e1 = _edge_embed(edge_attr, lin_e1_W, lin_e1_b)
  e2 = _edge_embed(edge_attr, lin_e2_W, lin_e2_b)

  agg1 = _make_edge_pass(D)(x, e1, src, dst)
  h1 = _node_mlp(x, agg1, nn1_W1, nn1_b1, nn1_W2, nn1_b2, relu_out=True)

  agg2 = _make_edge_pass(64)(h1, e2, src, dst)
  h2 = _node_mlp(h1, agg2, nn2_W1, nn2_b1, nn2_W2, nn2_b2, relu_out=False)

  sums, cnt = _pool(h2, batch)
  return _head(sums, cnt, usr, m1_W, m1_b, m2_W, m2_b, m3_W, m3_b,
               m4_W, m4_b, m5_W, m5_b)
